# baseline (speedup 1.0000x reference)
import jax
import jax.numpy as jnp
from jax.experimental import pallas as pl
from jax.experimental.pallas import tpu as pltpu

M = 10000
K = 10000
N = 128
BM = 400
NBUF = 3
NSTEPS = M // BM


def _body(adj_hbm, x_hbm, out_ref, buf, xf, xb, sems, xsem):
    i = pl.program_id(0)

    def adj_copy(block, slot):
        return pltpu.make_async_copy(
            adj_hbm.at[pl.ds(block * BM, BM), :], buf.at[slot], sems.at[slot]
        )

    @pl.when(i == 0)
    def _prologue():
        for j in range(NBUF - 1):
            adj_copy(j, j).start()
        pltpu.make_async_copy(x_hbm, xf, xsem).start()

    @pl.when(i + NBUF - 1 < NSTEPS)
    def _prefetch():
        block = i + NBUF - 1
        adj_copy(block, block % NBUF).start()

    slot = jax.lax.rem(i, NBUF)
    adj_copy(i, slot).wait()

    @pl.when(i == 0)
    def _x_once():
        pltpu.make_async_copy(x_hbm, xf, xsem).wait()
        xb[...] = xf[...].astype(jnp.bfloat16)

    out_ref[...] = jnp.dot(
        buf[slot].astype(jnp.bfloat16), xb[...], preferred_element_type=jnp.float32
    )


def kernel(x, adj):
    return pl.pallas_call(
        _body,
        grid=(NSTEPS,),
        in_specs=[
            pl.BlockSpec(memory_space=pl.ANY),
            pl.BlockSpec(memory_space=pl.ANY),
        ],
        out_specs=pl.BlockSpec((BM, N), lambda i: (i, 0)),
        out_shape=jax.ShapeDtypeStruct((M, N), jnp.float32),
        scratch_shapes=[
            pltpu.VMEM((NBUF, BM, K), jnp.float32),
            pltpu.VMEM((K, N), jnp.float32),
            pltpu.VMEM((K, N), jnp.bfloat16),
            pltpu.SemaphoreType.DMA((NBUF,)),
            pltpu.SemaphoreType.DMA,
        ],
        compiler_params=pltpu.CompilerParams(
            dimension_semantics=("arbitrary",),
        ),
    )(adj, x)


# speedup vs baseline: 1.0291x; 1.0291x over previous
import jax
import jax.numpy as jnp
from jax.experimental import pallas as pl
from jax.experimental.pallas import tpu as pltpu

M = 10000
K = 10000
N = 128
BM = 400


def _matmul_block(adj_ref, x_ref, out_ref):
    out_ref[...] = jax.lax.dot_general(
        adj_ref[...],
        x_ref[...],
        (((1,), (0,)), ((), ())),
        precision=jax.lax.Precision.DEFAULT,
        preferred_element_type=jnp.float32,
    )


def kernel(x, adj):
    return pl.pallas_call(
        _matmul_block,
        grid=(M // BM,),
        in_specs=[
            pl.BlockSpec((BM, K), lambda i: (i, 0)),
            pl.BlockSpec((K, N), lambda i: (0, 0)),
        ],
        out_specs=pl.BlockSpec((BM, N), lambda i: (i, 0)),
        out_shape=jax.ShapeDtypeStruct((M, N), jnp.float32),
        compiler_params=pltpu.CompilerParams(
            dimension_semantics=("arbitrary",),
        ),
    )(adj, x)
